# baseline (speedup 1.0000x reference)
import jax
import jax.numpy as jnp
from jax import lax
from jax.experimental import pallas as pl
from jax.experimental.pallas import tpu as pltpu
from jax.experimental.pallas import tpu_sc as plsc

_M = 7
_N = 16
_A = 8192
_TK = 250
_NG = 2
_NWG = 16
_AS = _A // _NWG
_TKW = _TK // _NG
_NRB = 2
_NQ = _N // _NRB


def _sc_body(le_tab, idx_hbm, mult_hbm, out_hbm,
             idx_v, mult_v, ab_bufs, o_bufs,
             sem_g, sem_o0, sem_o1):
    wid = lax.axis_index("s") * 2 + lax.axis_index("c")
    g = wid // _NWG
    a0 = (wid % _NWG) * _AS
    pltpu.sync_copy(idx_hbm.at[wid], idx_v)
    pltpu.sync_copy(mult_hbm.at[g], mult_v)
    sem_o = (sem_o0, sem_o1)

    def issue_gather(t, slot):
        idx = idx_v.at[pl.ds(t * 2 * _N, 2 * _N)]
        pltpu.async_copy(le_tab.at[idx], ab_bufs.at[slot], sem_g)

    def wait_gather(slot):
        pltpu.make_async_copy(
            le_tab.at[pl.ds(0, 2 * _N)], ab_bufs.at[slot], sem_g
        ).wait()

    def out_slice(tk, q):
        return out_hbm.at[tk, pl.ds(q * _NRB, _NRB), :, pl.ds(a0, _AS)]

    def wait_scatter(oslot, tk, q):
        pltpu.make_async_copy(o_bufs.at[oslot], out_slice(tk, q), sem_o[oslot]).wait()

    def compute_quarter(gslot, q, mv):
        i0 = q * _NRB
        oslot = q % 2

        def c_body(c, _):
            cs = pl.ds(c * 16, 16)
            a_regs = [ab_bufs[gslot, i0 + i, cs] for i in range(_NRB)]
            for j in range(_N):
                bmj = ab_bufs[gslot, _N + j, cs] * mv
                for i in range(_NRB):
                    o_bufs[oslot, i, j, cs] = a_regs[i] * bmj
            return 0

        lax.fori_loop(0, _AS // 16, c_body, 0)

    def process(t, gslot, first, issue_after=None):
        tk = _NG * t + g
        wait_gather(gslot)
        if issue_after is not None:
            issue_after()
        mv = mult_v[t]
        for q in range(_NQ):
            oslot = q % 2
            if q >= 2:
                wait_scatter(oslot, tk, q)
            elif first is None:
                wait_scatter(oslot, tk, q)
            else:
                @pl.when(jnp.logical_not(first))
                def _():
                    wait_scatter(oslot, tk, q)
            compute_quarter(gslot, q, mv)
            pltpu.async_copy(o_bufs.at[oslot], out_slice(tk, q), sem_o[oslot])

    issue_gather(0, 0)

    @pl.loop(0, _TKW - 1, step=2)
    def pair(t0):
        process(t0, 0, first=(t0 == 0),
                issue_after=lambda: issue_gather(t0 + 1, 1))

        def _issue_next_pair():
            @pl.when(t0 + 2 < _TKW)
            def _():
                issue_gather(t0 + 2, 0)

        process(t0 + 1, 1, first=None, issue_after=_issue_next_pair)

    process(_TKW - 1, 0, first=None)

    wait_scatter(0, _NG * (_TKW - 1) + g, _NQ - 2)
    wait_scatter(1, _NG * (_TKW - 1) + g, _NQ - 1)


def kernel(LE_1, indices_prev, indices_1, l_tuples, multiplicities_t):
    T, K = indices_prev.shape
    rows_a = (l_tuples[:, 0][:, None] * _M + indices_prev).reshape(-1)
    rows_b = (l_tuples[:, 1][:, None] * _M + indices_1).reshape(-1)
    rows_a = rows_a.astype(jnp.int32)
    rows_b = rows_b.astype(jnp.int32)
    n_off = jnp.arange(_N, dtype=jnp.int32)[None, :] * _NWG
    idx_a = rows_a[:, None] * (_N * _NWG) + n_off
    idx_b = rows_b[:, None] * (_N * _NWG) + n_off
    idx_tk = jnp.concatenate([idx_a, idx_b], axis=1)
    c_off = jnp.arange(_NWG, dtype=jnp.int32)[:, None]
    idx_ab = jnp.concatenate(
        [idx_tk[g::_NG].reshape(1, -1) + c_off for g in range(_NG)], axis=0
    )
    mult_flat = multiplicities_t.reshape(-1)
    mult_b = jnp.stack(
        [
            jnp.broadcast_to(mult_flat[gg::_NG][:, None], (_TKW, _N))
            for gg in range(_NG)
        ],
        axis=0,
    ).astype(jnp.float32)
    le_tab = LE_1.reshape(-1, _AS)

    mesh = plsc.VectorSubcoreMesh(core_axis_name="c", subcore_axis_name="s")
    sck = pl.kernel(
        _sc_body,
        out_type=jax.ShapeDtypeStruct((_TK, _N, _N, _A), jnp.float32),
        mesh=mesh,
        scratch_types=[
            pltpu.VMEM((_TKW * 2 * _N,), jnp.int32),
            pltpu.VMEM((_TKW, _N), jnp.float32),
            pltpu.VMEM((2, 2 * _N, _AS), jnp.float32),
            pltpu.VMEM((2, _NRB, _N, _AS), jnp.float32),
            pltpu.SemaphoreType.DMA,
            pltpu.SemaphoreType.DMA,
            pltpu.SemaphoreType.DMA,
        ],
    )
    out = sck(le_tab, idx_ab, mult_b)
    return out.reshape(T, K, _N * _N, _A)


# speedup vs baseline: 1.5704x; 1.5704x over previous
import jax
import jax.numpy as jnp
from jax import lax
from jax.experimental import pallas as pl
from jax.experimental.pallas import tpu as pltpu
from jax.experimental.pallas import tpu_sc as plsc

_M = 7
_N = 16
_A = 8192
_TK = 250
_NW = 32
_AS = _A // _NW
_NH = 2
_NR = _N // _NH


def _sc_body(le_tab, idx_hbm, mult_hbm, out_hbm,
             idx_v, mult_v, ab_bufs, o_bufs,
             sem_g, sem_o0, sem_o1):
    wid = lax.axis_index("s") * 2 + lax.axis_index("c")
    pltpu.sync_copy(idx_hbm.at[wid], idx_v)
    pltpu.sync_copy(mult_hbm, mult_v)
    a0 = wid * _AS
    sem_o = (sem_o0, sem_o1)

    def issue_gather(tk, slot):
        idx = idx_v.at[pl.ds(tk * 2 * _N, 2 * _N)]
        pltpu.async_copy(le_tab.at[idx], ab_bufs.at[slot], sem_g)

    def wait_gather(slot):
        pltpu.make_async_copy(
            le_tab.at[pl.ds(0, 2 * _N)], ab_bufs.at[slot], sem_g
        ).wait()

    def out_slice(tk, h):
        return out_hbm.at[tk, pl.ds(h * _NR, _NR), :, pl.ds(a0, _AS)]

    def wait_scatter(h, tk):
        pltpu.make_async_copy(o_bufs.at[h], out_slice(tk, h), sem_o[h]).wait()

    def compute_half(slot, h, mv):
        i0 = h * _NR

        def c_body(c, _):
            cs = pl.ds(c * 16, 16)
            a_regs = [ab_bufs[slot, i0 + i, cs] for i in range(_NR)]
            for j in range(_N):
                bmj = ab_bufs[slot, _N + j, cs] * mv
                for i in range(_NR):
                    o_bufs[h, i, j, cs] = a_regs[i] * bmj
            return 0

        lax.fori_loop(0, _AS // 16, c_body, 0)

    issue_gather(0, 0)

    @pl.loop(0, _TK, step=2)
    def pair(tk0):
        for s in range(2):
            tk = tk0 + s
            wait_gather(slot=s)
            if s == 0:
                issue_gather(tk0 + 1, 1)
            else:
                @pl.when(tk0 + 2 < _TK)
                def _():
                    issue_gather(tk0 + 2, 0)
            mv = mult_v[tk]
            for h in range(_NH):
                if s == 0:
                    @pl.when(tk0 > 0)
                    def _():
                        wait_scatter(h, tk)
                else:
                    wait_scatter(h, tk)
                compute_half(s, h, mv)
                pltpu.async_copy(o_bufs.at[h], out_slice(tk, h), sem_o[h])

    for h in range(_NH):
        wait_scatter(h, _TK - 1)


def kernel(LE_1, indices_prev, indices_1, l_tuples, multiplicities_t):
    T, K = indices_prev.shape
    rows_a = (l_tuples[:, 0][:, None] * _M + indices_prev).reshape(-1)
    rows_b = (l_tuples[:, 1][:, None] * _M + indices_1).reshape(-1)
    rows_a = rows_a.astype(jnp.int32)
    rows_b = rows_b.astype(jnp.int32)
    n_off = jnp.arange(_N, dtype=jnp.int32)[None, :] * _NW
    w_off = jnp.arange(_NW, dtype=jnp.int32)[:, None, None]
    idx_a = rows_a[:, None] * (_N * _NW) + n_off
    idx_b = rows_b[:, None] * (_N * _NW) + n_off
    idx_ab = jnp.concatenate([idx_a, idx_b], axis=1)[None] + w_off
    idx_ab = idx_ab.reshape(_NW, _TK * 2 * _N)
    mult_b = jnp.broadcast_to(
        multiplicities_t.reshape(-1)[:, None], (_TK, _N)
    ).astype(jnp.float32)
    le_tab = LE_1.reshape(-1, _AS)

    mesh = plsc.VectorSubcoreMesh(core_axis_name="c", subcore_axis_name="s")
    sck = pl.kernel(
        _sc_body,
        out_type=jax.ShapeDtypeStruct((_TK, _N, _N, _A), jnp.float32),
        mesh=mesh,
        scratch_types=[
            pltpu.VMEM((_TK * 2 * _N,), jnp.int32),
            pltpu.VMEM((_TK, _N), jnp.float32),
            pltpu.VMEM((2, 2 * _N, _AS), jnp.float32),
            pltpu.VMEM((_NH, _NR, _N, _AS), jnp.float32),
            pltpu.SemaphoreType.DMA,
            pltpu.SemaphoreType.DMA,
            pltpu.SemaphoreType.DMA,
        ],
    )
    out = sck(le_tab, idx_ab, mult_b)
    return out.reshape(T, K, _N * _N, _A)


# speedup vs baseline: 1.5781x; 1.0049x over previous
import jax
import jax.numpy as jnp
from jax import lax
from jax.experimental import pallas as pl
from jax.experimental.pallas import tpu as pltpu
from jax.experimental.pallas import tpu_sc as plsc

_M = 7
_N = 16
_A = 8192
_TK = 250
_NW = 32
_AS = _A // _NW
_NH = 2
_NR = _N // _NH


def _sc_body(le_tab, idx_hbm, mult_hbm, out_hbm,
             idx_v, mult_v, ab_bufs, o_bufs,
             sem_g, sem_o0, sem_o1):
    wid = lax.axis_index("s") * 2 + lax.axis_index("c")
    pltpu.sync_copy(idx_hbm.at[wid], idx_v)
    pltpu.sync_copy(mult_hbm, mult_v)
    a0 = wid * _AS
    sem_o = (sem_o0, sem_o1)

    def issue_gather(tk, slot):
        idx = idx_v.at[pl.ds(tk * 2 * _N, 2 * _N)]
        pltpu.async_copy(le_tab.at[idx], ab_bufs.at[slot], sem_g)

    def wait_gather(slot):
        pltpu.make_async_copy(
            le_tab.at[pl.ds(0, 2 * _N)], ab_bufs.at[slot], sem_g
        ).wait()

    def out_slice(tk, h):
        return out_hbm.at[tk, pl.ds(h * _NR, _NR), :, pl.ds(a0, _AS)]

    def wait_scatter(h, tk):
        pltpu.make_async_copy(o_bufs.at[h], out_slice(tk, h), sem_o[h]).wait()

    def compute_half(slot, h, mv):
        i0 = h * _NR

        @plsc.parallel_loop(0, _AS // 16, step=1, unroll=2)
        def c_body(c):
            cs = pl.ds(c * 16, 16)
            a_regs = [ab_bufs[slot, i0 + i, cs] for i in range(_NR)]
            for j in range(_N):
                bmj = ab_bufs[slot, _N + j, cs] * mv
                for i in range(_NR):
                    o_bufs[h, i, j, cs] = a_regs[i] * bmj

    issue_gather(0, 0)

    @pl.loop(0, _TK, step=2)
    def pair(tk0):
        for s in range(2):
            tk = tk0 + s
            wait_gather(slot=s)
            if s == 0:
                issue_gather(tk0 + 1, 1)
            else:
                @pl.when(tk0 + 2 < _TK)
                def _():
                    issue_gather(tk0 + 2, 0)
            mv = mult_v[tk]
            for h in range(_NH):
                if s == 0:
                    @pl.when(tk0 > 0)
                    def _():
                        wait_scatter(h, tk)
                else:
                    wait_scatter(h, tk)
                compute_half(s, h, mv)
                pltpu.async_copy(o_bufs.at[h], out_slice(tk, h), sem_o[h])

    for h in range(_NH):
        wait_scatter(h, _TK - 1)


def kernel(LE_1, indices_prev, indices_1, l_tuples, multiplicities_t):
    T, K = indices_prev.shape
    rows_a = (l_tuples[:, 0][:, None] * _M + indices_prev).reshape(-1)
    rows_b = (l_tuples[:, 1][:, None] * _M + indices_1).reshape(-1)
    rows_a = rows_a.astype(jnp.int32)
    rows_b = rows_b.astype(jnp.int32)
    n_off = jnp.arange(_N, dtype=jnp.int32)[None, :] * _NW
    w_off = jnp.arange(_NW, dtype=jnp.int32)[:, None, None]
    idx_a = rows_a[:, None] * (_N * _NW) + n_off
    idx_b = rows_b[:, None] * (_N * _NW) + n_off
    idx_ab = jnp.concatenate([idx_a, idx_b], axis=1)[None] + w_off
    idx_ab = idx_ab.reshape(_NW, _TK * 2 * _N)
    mult_b = jnp.broadcast_to(
        multiplicities_t.reshape(-1)[:, None], (_TK, _N)
    ).astype(jnp.float32)
    le_tab = LE_1.reshape(-1, _AS)

    mesh = plsc.VectorSubcoreMesh(core_axis_name="c", subcore_axis_name="s")
    sck = pl.kernel(
        _sc_body,
        out_type=jax.ShapeDtypeStruct((_TK, _N, _N, _A), jnp.float32),
        mesh=mesh,
        scratch_types=[
            pltpu.VMEM((_TK * 2 * _N,), jnp.int32),
            pltpu.VMEM((_TK, _N), jnp.float32),
            pltpu.VMEM((2, 2 * _N, _AS), jnp.float32),
            pltpu.VMEM((_NH, _NR, _N, _AS), jnp.float32),
            pltpu.SemaphoreType.DMA,
            pltpu.SemaphoreType.DMA,
            pltpu.SemaphoreType.DMA,
        ],
    )
    out = sck(le_tab, idx_ab, mult_b)
    return out.reshape(T, K, _N * _N, _A)
